# Initial kernel scaffold; baseline (speedup 1.0000x reference)
import jax
import jax.numpy as jnp
from jax.experimental import pallas as pl


def kernel(x, edge_index, W1, b1, W2, b2, Wf, bf, Ws, bs):
    raise NotImplementedError("write your pallas kernel here")


# speedup vs baseline: 12.8621x; 12.8621x over previous
import functools

import jax
import jax.numpy as jnp
from jax import lax
from jax.experimental import pallas as pl
from jax.experimental.pallas import tpu as pltpu
from jax.experimental.pallas import tpu_sc as plsc

NC, NS, LANES = 2, 16, 16
NW = NC * NS
CH = 128
ZR = 64


def _edge_aggregate(g, src3, dst3, zeros2d, n_pad):
    n_chunks = src3.shape[1]
    d = g.shape[1]
    rows_per_tile = n_pad // NS

    mesh = plsc.VectorSubcoreMesh(core_axis_name="c", subcore_axis_name="s")

    @functools.partial(
        pl.kernel,
        out_type=jax.ShapeDtypeStruct((NC, n_pad, d), jnp.float32),
        mesh=mesh,
        scratch_types=[
            pltpu.VMEM((n_chunks, CH), jnp.int32),
            pltpu.VMEM((n_chunks, CH), jnp.int32),
            pltpu.VMEM((CH, d), jnp.float32),
            pltpu.VMEM((ZR, d), jnp.float32),
            pltpu.VMEM_SHARED((n_pad, d), jnp.float32),
            pltpu.SemaphoreType.DMA,
        ],
    )
    def body(g_hbm, src_hbm, dst_hbm, z_hbm, out_hbm, srcv, dstv, buf, zv, acc, sem):
        c = lax.axis_index("c")
        s = lax.axis_index("s")
        wid = c * NS + s
        pltpu.sync_copy(z_hbm, zv)
        pltpu.sync_copy(src_hbm.at[wid], srcv)
        pltpu.sync_copy(dst_hbm.at[wid], dstv)
        base = s * rows_per_tile
        for k in range(rows_per_tile // ZR):
            pltpu.sync_copy(zv, acc.at[pl.ds(base + k * ZR, ZR)])
        plsc.subcore_barrier()

        def chunk(j, carry):
            pltpu.async_copy(g_hbm.at[srcv.at[j]], buf, sem).wait()
            pltpu.sync_copy(buf, acc.at[dstv.at[j]], add=True)
            return carry

        lax.fori_loop(0, n_chunks, chunk, 0)
        plsc.subcore_barrier()
        pltpu.sync_copy(acc.at[pl.ds(base, rows_per_tile)],
                        out_hbm.at[c, pl.ds(base, rows_per_tile)])

    return body(g, src3, dst3, zeros2d)


def _degree(dst3, zeros1d, ones1d, n_pad):
    n_chunks = dst3.shape[1]
    rows_per_tile = n_pad // NS

    mesh = plsc.VectorSubcoreMesh(core_axis_name="c", subcore_axis_name="s")

    @functools.partial(
        pl.kernel,
        out_type=jax.ShapeDtypeStruct((NC, n_pad), jnp.float32),
        mesh=mesh,
        scratch_types=[
            pltpu.VMEM((n_chunks, CH), jnp.int32),
            pltpu.VMEM((CH,), jnp.float32),
            pltpu.VMEM((rows_per_tile,), jnp.float32),
            pltpu.VMEM_SHARED((n_pad,), jnp.float32),
        ],
    )
    def body(dst_hbm, z_hbm, ones_hbm, out_hbm, dstv, onesv, zv, acc):
        c = lax.axis_index("c")
        s = lax.axis_index("s")
        wid = c * NS + s
        pltpu.sync_copy(z_hbm, zv)
        pltpu.sync_copy(ones_hbm, onesv)
        pltpu.sync_copy(dst_hbm.at[wid], dstv)
        base = s * rows_per_tile
        pltpu.sync_copy(zv, acc.at[pl.ds(base, rows_per_tile)])
        plsc.subcore_barrier()

        def chunk(j, carry):
            pltpu.sync_copy(onesv, acc.at[dstv.at[j]], add=True)
            return carry

        lax.fori_loop(0, n_chunks, chunk, 0)
        plsc.subcore_barrier()
        pltpu.sync_copy(acc.at[pl.ds(base, rows_per_tile)],
                        out_hbm.at[c, pl.ds(base, rows_per_tile)])

    return body(dst3, zeros1d, ones1d)


_BR = 1000


def _tc_prep(x, W1, degp):
    n, d = x.shape

    def body(x_ref, w_ref, degp_ref, g_ref, dinv_ref):
        deg = degp_ref[0] + degp_ref[1] + 1.0
        dinv = 1.0 / jnp.sqrt(deg)
        dinv_ref[...] = dinv
        g_ref[...] = jnp.dot(x_ref[...], w_ref[...],
                             preferred_element_type=jnp.float32) * dinv

    return pl.pallas_call(
        body,
        grid=(n // _BR,),
        in_specs=[
            pl.BlockSpec((_BR, d), lambda i: (i, 0)),
            pl.BlockSpec((d, d), lambda i: (0, 0)),
            pl.BlockSpec((NC, _BR, 1), lambda i: (0, i, 0)),
        ],
        out_specs=[
            pl.BlockSpec((_BR, d), lambda i: (i, 0)),
            pl.BlockSpec((_BR, 1), lambda i: (i, 0)),
        ],
        out_shape=[
            jax.ShapeDtypeStruct((n, d), jnp.float32),
            jax.ShapeDtypeStruct((n, 1), jnp.float32),
        ],
    )(x, W1, degp)


def _tc_mid(parts, g1, dinv, b1, W2):
    n, d = g1.shape

    def body(p_ref, g_ref, dinv_ref, b_ref, w_ref, out_ref):
        sm = p_ref[0] + p_ref[1] + g_ref[...]
        x2 = jnp.maximum(sm * dinv_ref[...] + b_ref[...], 0.0)
        out_ref[...] = jnp.dot(x2, w_ref[...],
                               preferred_element_type=jnp.float32) * dinv_ref[...]

    return pl.pallas_call(
        body,
        grid=(n // _BR,),
        in_specs=[
            pl.BlockSpec((NC, _BR, d), lambda i: (0, i, 0)),
            pl.BlockSpec((_BR, d), lambda i: (i, 0)),
            pl.BlockSpec((_BR, 1), lambda i: (i, 0)),
            pl.BlockSpec((1, d), lambda i: (0, 0)),
            pl.BlockSpec((d, d), lambda i: (0, 0)),
        ],
        out_specs=pl.BlockSpec((_BR, d), lambda i: (i, 0)),
        out_shape=jax.ShapeDtypeStruct((n, d), jnp.float32),
    )(parts, g1, dinv, b1, W2)


def _tc_final(parts, g2, dinv, b2, Wf, x, Ws, bf, bs):
    n, d = g2.shape
    dout = Wf.shape[1]

    def body(p_ref, g_ref, dinv_ref, b2_ref, wf_ref, x_ref, ws_ref,
             bf_ref, bs_ref, out_ref):
        sm = p_ref[0] + p_ref[1] + g_ref[...]
        h2 = jnp.maximum(sm * dinv_ref[...] + b2_ref[...], 0.0)
        out_ref[...] = (jnp.dot(h2, wf_ref[...], preferred_element_type=jnp.float32)
                        + jnp.dot(x_ref[...], ws_ref[...],
                                  preferred_element_type=jnp.float32)
                        + bf_ref[...] + bs_ref[...])

    return pl.pallas_call(
        body,
        grid=(n // _BR,),
        in_specs=[
            pl.BlockSpec((NC, _BR, d), lambda i: (0, i, 0)),
            pl.BlockSpec((_BR, d), lambda i: (i, 0)),
            pl.BlockSpec((_BR, 1), lambda i: (i, 0)),
            pl.BlockSpec((1, d), lambda i: (0, 0)),
            pl.BlockSpec((d, dout), lambda i: (0, 0)),
            pl.BlockSpec((_BR, x.shape[1]), lambda i: (i, 0)),
            pl.BlockSpec((x.shape[1], dout), lambda i: (0, 0)),
            pl.BlockSpec((1, dout), lambda i: (0, 0)),
            pl.BlockSpec((1, dout), lambda i: (0, 0)),
        ],
        out_specs=pl.BlockSpec((_BR, dout), lambda i: (i, 0)),
        out_shape=jax.ShapeDtypeStruct((n, dout), jnp.float32),
    )(parts, g2, dinv, b2, Wf, x, Ws, bf, bs)


def kernel(x, edge_index, W1, b1, W2, b2, Wf, bf, Ws, bs):
    n, d = x.shape
    e = edge_index.shape[1]

    per_tile = -(-e // (NW * CH)) * CH
    e_pad = per_tile * NW
    n_chunks = per_tile // CH
    ei = edge_index.astype(jnp.int32)
    src3 = jnp.pad(ei[0], (0, e_pad - e)).reshape(NW, n_chunks, CH)
    dst3 = jnp.pad(ei[1], (0, e_pad - e),
                   constant_values=n).reshape(NW, n_chunks, CH)

    n_pad = -(-n // (NS * ZR)) * (NS * ZR)
    zeros2d = jnp.zeros((ZR, d), jnp.float32)
    zeros1d = jnp.zeros((n_pad // NS,), jnp.float32)
    ones1d = jnp.ones((CH,), jnp.float32)

    degp = _degree(dst3, zeros1d, ones1d, n_pad)
    degp3 = degp.reshape(NC, n_pad, 1)
    g1, dinv = _tc_prep(x, W1, degp3)
    parts1 = _edge_aggregate(g1, src3, dst3, zeros2d, n_pad)
    g2 = _tc_mid(parts1, g1, dinv, b1.reshape(1, -1), W2)
    parts2 = _edge_aggregate(g2, src3, dst3, zeros2d, n_pad)
    return _tc_final(parts2, g2, dinv, b2.reshape(1, -1), Wf, x, Ws,
                     bf.reshape(1, -1), bs.reshape(1, -1))
